# baseline (device time: 448467 ns/iter reference)
import jax
import jax.numpy as jnp
from jax import lax
from jax.experimental import pallas as pl
from jax.experimental.pallas import tpu as pltpu

N_DEV = 32
N_EXPERTS = 128
CAPACITY = 102.0


def kernel(x, router_W, route_idx, expert_W):
    del router_W
    tokens, d = x.shape
    e_loc, _, h = expert_W.shape

    def body(x_ref, ridx_ref, ew_ref, out_ref,
             comm_w, comm_c, send_w, recv_w, send_c, recv_c):
        my = lax.axis_index("i")
        left = lax.rem(my + (N_DEV - 1), N_DEV)
        right = lax.rem(my + 1, N_DEV)

        barrier = pltpu.get_barrier_semaphore()
        for nbr in (left, right):
            pl.semaphore_signal(barrier, inc=1, device_id=(nbr,),
                                device_id_type=pl.DeviceIdType.MESH)
        pl.semaphore_wait(barrier, 2)

        x_bf = x_ref[:, :].astype(jnp.bfloat16)
        route = ridx_ref[:, :]

        eids = lax.broadcasted_iota(jnp.int32, (tokens, N_EXPERTS), 1)
        oh = (route == eids).astype(jnp.float32)
        row = lax.broadcasted_iota(jnp.int32, (tokens, tokens), 0)
        col = lax.broadcasted_iota(jnp.int32, (tokens, tokens), 1)
        lower = (col < row).astype(jnp.float32)
        csum_excl = jnp.dot(lower, oh, preferred_element_type=jnp.float32)
        rank = jnp.sum(csum_excl * oh, axis=1, keepdims=True)
        counts = jnp.sum(oh, axis=0, keepdims=True)

        comm_w[0, :, :, :] = ew_ref[:, :, :].astype(jnp.bfloat16)
        comm_c[0, :, :] = jnp.broadcast_to(
            counts.astype(jnp.int32), comm_c.shape[1:])

        out_ref[:, :] = jnp.zeros((tokens, h), jnp.float32)

        def accum(origin, slot):
            for j in range(e_loc):
                e = origin * e_loc + j
                m = (route == e).astype(jnp.bfloat16)
                out_ref[:, :] += jnp.dot(
                    x_bf * m, comm_w[slot, j, :, :],
                    preferred_element_type=jnp.float32)

        accum(my, 0)

        prefix = jnp.zeros((1, N_EXPERTS), jnp.float32)
        for hop in range(N_DEV - 1):
            s = hop % 2
            r = (hop + 1) % 2
            rw = pltpu.make_async_remote_copy(
                src_ref=comm_w.at[s], dst_ref=comm_w.at[r],
                send_sem=send_w.at[s], recv_sem=recv_w.at[r],
                device_id=(right,), device_id_type=pl.DeviceIdType.MESH)
            rc = pltpu.make_async_remote_copy(
                src_ref=comm_c.at[s], dst_ref=comm_c.at[r],
                send_sem=send_c.at[s], recv_sem=recv_c.at[r],
                device_id=(right,), device_id_type=pl.DeviceIdType.MESH)
            rw.start()
            rc.start()
            rw.wait()
            rc.wait()
            origin = lax.rem(my + (N_DEV - 1 - hop), N_DEV)
            cnts = comm_c[r, 0:1, :].astype(jnp.float32)
            prefix = prefix + jnp.where(origin < my, cnts, 0.0)
            accum(origin, r)

        my_prefix = jnp.sum(oh * prefix, axis=1, keepdims=True)
        accept = ((my_prefix + rank) < CAPACITY).astype(jnp.float32)
        out_ref[:, :] *= accept

    return pl.pallas_call(
        body,
        out_shape=jax.ShapeDtypeStruct((tokens, h), jnp.float32),
        in_specs=[pl.BlockSpec(memory_space=pltpu.VMEM)] * 3,
        out_specs=pl.BlockSpec(memory_space=pltpu.VMEM),
        scratch_shapes=[
            pltpu.VMEM((2, e_loc, d, h), jnp.bfloat16),
            pltpu.VMEM((2, 8, N_EXPERTS), jnp.int32),
            pltpu.SemaphoreType.DMA((2,)),
            pltpu.SemaphoreType.DMA((2,)),
            pltpu.SemaphoreType.DMA((2,)),
            pltpu.SemaphoreType.DMA((2,)),
        ],
        compiler_params=pltpu.CompilerParams(collective_id=0),
    )(x, route_idx, expert_W)


# device time: 402257 ns/iter; 1.1149x vs baseline; 1.1149x over previous
import functools

import jax
import jax.numpy as jnp
from jax import lax
from jax.experimental import pallas as pl
from jax.experimental.pallas import tpu as pltpu

N_DEV = 32
N_EXPERTS = 128
CAPACITY = 102.0
F_HOPS = 16
B_HOPS = 15
SLOTS_F = 9
SLOTS_B = 8


def kernel(x, router_W, route_idx, expert_W):
    del router_W
    tokens, d = x.shape
    e_loc, _, h = expert_W.shape

    def body(x_ref, ridx_ref, ew_ref, out_ref,
             comm_f, comm_b, comm_cf, comm_cb,
             sw_f, rw_f, sw_b, rw_b, sc_f, rc_f, sc_b, rc_b):
        my = lax.axis_index("i")
        left = lax.rem(my + (N_DEV - 1), N_DEV)
        right = lax.rem(my + 1, N_DEV)

        barrier = pltpu.get_barrier_semaphore()
        for nbr in (left, right):
            pl.semaphore_signal(barrier, inc=1, device_id=(nbr,),
                                device_id_type=pl.DeviceIdType.MESH)
        pl.semaphore_wait(barrier, 2)

        x_bf = x_ref[:, :].astype(jnp.bfloat16)
        route = ridx_ref[:, :]

        eids = lax.broadcasted_iota(jnp.int32, (tokens, N_EXPERTS), 1)
        oh = (route == eids).astype(jnp.float32)
        row = lax.broadcasted_iota(jnp.int32, (tokens, tokens), 0)
        col = lax.broadcasted_iota(jnp.int32, (tokens, tokens), 1)
        lower = (col < row).astype(jnp.float32)
        csum_excl = jnp.dot(lower, oh, preferred_element_type=jnp.float32)
        rank = jnp.sum(csum_excl * oh, axis=1, keepdims=True)
        counts = jnp.sum(oh, axis=0, keepdims=True)

        own_w = ew_ref[:, :, :].astype(jnp.bfloat16)
        comm_f[0, :, :, :] = own_w
        comm_b[0, :, :, :] = own_w
        cbcast = jnp.broadcast_to(counts.astype(jnp.int32), comm_cf.shape[1:])
        comm_cf[0, :, :] = cbcast
        comm_cb[0, :, :] = cbcast

        out_ref[:, :] = jnp.zeros((tokens, h), jnp.float32)

        def accum(origin, comm, slot):
            for j in range(e_loc):
                e = origin * e_loc + j
                m = (route == e).astype(jnp.bfloat16)
                out_ref[:, :] += jnp.dot(
                    x_bf * m, comm[slot, j, :, :],
                    preferred_element_type=jnp.float32)

        def pfx(prefix, origin, comm_c, slot):
            cnts = comm_c[slot, 0:1, :].astype(jnp.float32)
            return prefix + jnp.where(origin < my, cnts, 0.0)

        def mk(comm, comm_c, ssem, rsem, csem, crsem, dst, hop, n_slots):
            rw = pltpu.make_async_remote_copy(
                src_ref=comm.at[(hop - 1) % n_slots],
                dst_ref=comm.at[hop % n_slots],
                send_sem=ssem.at[hop - 1], recv_sem=rsem.at[hop - 1],
                device_id=(dst,), device_id_type=pl.DeviceIdType.MESH)
            rc = pltpu.make_async_remote_copy(
                src_ref=comm_c.at[hop - 1], dst_ref=comm_c.at[hop],
                send_sem=csem.at[hop - 1], recv_sem=crsem.at[hop - 1],
                device_id=(dst,), device_id_type=pl.DeviceIdType.MESH)
            return rw, rc

        fwd = [mk(comm_f, comm_cf, sw_f, rw_f, sc_f, rc_f, right, hp, SLOTS_F)
               for hp in range(1, F_HOPS + 1)]
        bwd = [mk(comm_b, comm_cb, sw_b, rw_b, sc_b, rc_b, left, hp, SLOTS_B)
               for hp in range(1, B_HOPS + 1)]

        for r_ in fwd[0] + bwd[0]:
            r_.start()

        accum(my, comm_f, 0)

        prefix = jnp.zeros((1, N_EXPERTS), jnp.float32)
        for hop in range(1, F_HOPS + 1):
            for r_ in fwd[hop - 1]:
                r_.wait_recv()
            if hop <= B_HOPS:
                for r_ in bwd[hop - 1]:
                    r_.wait_recv()
            if hop < F_HOPS:
                for r_ in fwd[hop]:
                    r_.start()
            if hop < B_HOPS:
                for r_ in bwd[hop]:
                    r_.start()

            of = lax.rem(my + (N_DEV - hop), N_DEV)
            accum(of, comm_f, hop % SLOTS_F)
            prefix = pfx(prefix, of, comm_cf, hop)
            if hop <= B_HOPS:
                ob = lax.rem(my + hop, N_DEV)
                accum(ob, comm_b, hop % SLOTS_B)
                prefix = pfx(prefix, ob, comm_cb, hop)

        my_prefix = jnp.sum(oh * prefix, axis=1, keepdims=True)
        accept = ((my_prefix + rank) < CAPACITY).astype(jnp.float32)
        out_ref[:, :] *= accept

        for pair in fwd + bwd:
            for r_ in pair:
                r_.wait_send()

        @functools.partial(pl.run_scoped,
                           second_barrier=pltpu.SemaphoreType.REGULAR)
        def _(second_barrier):
            for nbr in (left, right):
                pl.semaphore_signal(second_barrier, inc=1, device_id=(nbr,),
                                    device_id_type=pl.DeviceIdType.MESH)
            pl.semaphore_wait(second_barrier, 2)

    return pl.pallas_call(
        body,
        out_shape=jax.ShapeDtypeStruct((tokens, h), jnp.float32),
        in_specs=[pl.BlockSpec(memory_space=pltpu.VMEM)] * 3,
        out_specs=pl.BlockSpec(memory_space=pltpu.VMEM),
        scratch_shapes=[
            pltpu.VMEM((SLOTS_F, e_loc, d, h), jnp.bfloat16),
            pltpu.VMEM((SLOTS_B, e_loc, d, h), jnp.bfloat16),
            pltpu.VMEM((F_HOPS + 1, 8, N_EXPERTS), jnp.int32),
            pltpu.VMEM((B_HOPS + 1, 8, N_EXPERTS), jnp.int32),
            pltpu.SemaphoreType.DMA((F_HOPS,)),
            pltpu.SemaphoreType.DMA((F_HOPS,)),
            pltpu.SemaphoreType.DMA((B_HOPS,)),
            pltpu.SemaphoreType.DMA((B_HOPS,)),
            pltpu.SemaphoreType.DMA((F_HOPS,)),
            pltpu.SemaphoreType.DMA((F_HOPS,)),
            pltpu.SemaphoreType.DMA((B_HOPS,)),
            pltpu.SemaphoreType.DMA((B_HOPS,)),
        ],
        compiler_params=pltpu.CompilerParams(collective_id=0),
    )(x, route_idx, expert_W)


# device time: 228002 ns/iter; 1.9669x vs baseline; 1.7643x over previous
import functools

import jax
import jax.numpy as jnp
from jax import lax
from jax.experimental import pallas as pl
from jax.experimental.pallas import tpu as pltpu

N_DEV = 32
N_EXPERTS = 128
CAPACITY = 102.0
F_HOPS = 16
B_HOPS = 15
SLOTS_F = 9
SLOTS_B = 8

_POS = {}
_p = 0
for _z in range(4):
    for _y in range(4):
        for _x in ([0, 1] if _y % 2 == 0 else [1, 0]):
            _POS[(_x, _y, _z)] = _p
            _p += 1

_SNAKE_YZ = [(0, 0), (1, 0), (2, 0), (3, 0), (3, 1), (2, 1), (1, 1), (0, 1),
             (0, 2), (1, 2), (2, 2), (3, 2), (3, 3), (2, 3), (1, 3), (0, 3)]
_CYC = ([(0, y, z) for (y, z) in _SNAKE_YZ]
        + [(1, y, z) for (y, z) in reversed(_SNAKE_YZ)])
RING = [_POS[c] for c in _CYC]
SIGMA = [RING.index(m) for m in range(N_DEV)]


def kernel(x, router_W, route_idx, expert_W):
    del router_W
    tokens, d = x.shape
    e_loc, _, h = expert_W.shape

    my = lax.axis_index("i")
    ring = jnp.asarray(RING, jnp.int32)
    sigma = jnp.asarray(SIGMA, jnp.int32)
    ci = sigma[my]
    nxt = ring[(ci + 1) % N_DEV]
    prv = ring[(ci - 1) % N_DEV]
    ofs = ring[(ci - jnp.arange(1, F_HOPS + 1)) % N_DEV]
    obs = ring[(ci + jnp.arange(1, B_HOPS + 1)) % N_DEV]
    meta = jnp.concatenate(
        [jnp.stack([nxt, prv]), ofs, obs]).astype(jnp.int32).reshape(1, -1)

    def body(x_ref, ridx_ref, ew_ref, meta_ref, out_ref,
             comm_f, comm_b, comm_cf, comm_cb,
             sw_f, rw_f, sw_b, rw_b, sc_f, rc_f, sc_b, rc_b):
        my = lax.axis_index("i")
        right = meta_ref[0, 0]
        left = meta_ref[0, 1]

        def origin_f(hop):
            return meta_ref[0, 2 + (hop - 1)]

        def origin_b(hop):
            return meta_ref[0, 2 + F_HOPS + (hop - 1)]

        barrier = pltpu.get_barrier_semaphore()
        for nbr in (left, right):
            pl.semaphore_signal(barrier, inc=1, device_id=(nbr,),
                                device_id_type=pl.DeviceIdType.MESH)
        pl.semaphore_wait(barrier, 2)

        x_bf = x_ref[:, :].astype(jnp.bfloat16)
        route = ridx_ref[:, :]

        eids = lax.broadcasted_iota(jnp.int32, (tokens, N_EXPERTS), 1)
        oh = (route == eids).astype(jnp.float32)
        row = lax.broadcasted_iota(jnp.int32, (tokens, tokens), 0)
        col = lax.broadcasted_iota(jnp.int32, (tokens, tokens), 1)
        lower = (col < row).astype(jnp.float32)
        csum_excl = jnp.dot(lower, oh, preferred_element_type=jnp.float32)
        rank = jnp.sum(csum_excl * oh, axis=1, keepdims=True)
        counts = jnp.sum(oh, axis=0, keepdims=True)

        own_w = ew_ref[:, :, :].astype(jnp.bfloat16)
        comm_f[0, :, :, :] = own_w
        comm_b[0, :, :, :] = own_w
        cbcast = jnp.broadcast_to(counts.astype(jnp.int32), comm_cf.shape[1:])
        comm_cf[0, :, :] = cbcast
        comm_cb[0, :, :] = cbcast

        out_ref[:, :] = jnp.zeros((tokens, h), jnp.float32)

        def accum(origin, comm, slot):
            for j in range(e_loc):
                e = origin * e_loc + j
                m = (route == e).astype(jnp.bfloat16)
                out_ref[:, :] += jnp.dot(
                    x_bf * m, comm[slot, j, :, :],
                    preferred_element_type=jnp.float32)

        def pfx(prefix, origin, comm_c, slot):
            cnts = comm_c[slot, 0:1, :].astype(jnp.float32)
            return prefix + jnp.where(origin < my, cnts, 0.0)

        def mk(comm, comm_c, ssem, rsem, csem, crsem, dst, hop, n_slots):
            rw = pltpu.make_async_remote_copy(
                src_ref=comm.at[(hop - 1) % n_slots],
                dst_ref=comm.at[hop % n_slots],
                send_sem=ssem.at[hop - 1], recv_sem=rsem.at[hop - 1],
                device_id=(dst,), device_id_type=pl.DeviceIdType.MESH)
            rc = pltpu.make_async_remote_copy(
                src_ref=comm_c.at[hop - 1], dst_ref=comm_c.at[hop],
                send_sem=csem.at[hop - 1], recv_sem=crsem.at[hop - 1],
                device_id=(dst,), device_id_type=pl.DeviceIdType.MESH)
            return rw, rc

        fwd = [mk(comm_f, comm_cf, sw_f, rw_f, sc_f, rc_f, right, hp, SLOTS_F)
               for hp in range(1, F_HOPS + 1)]
        bwd = [mk(comm_b, comm_cb, sw_b, rw_b, sc_b, rc_b, left, hp, SLOTS_B)
               for hp in range(1, B_HOPS + 1)]

        for r_ in fwd[0] + bwd[0]:
            r_.start()

        accum(my, comm_f, 0)

        prefix = jnp.zeros((1, N_EXPERTS), jnp.float32)
        for hop in range(1, F_HOPS + 1):
            for r_ in fwd[hop - 1]:
                r_.wait_recv()
            if hop <= B_HOPS:
                for r_ in bwd[hop - 1]:
                    r_.wait_recv()
            if hop < F_HOPS:
                for r_ in fwd[hop]:
                    r_.start()
            if hop < B_HOPS:
                for r_ in bwd[hop]:
                    r_.start()

            of = origin_f(hop)
            accum(of, comm_f, hop % SLOTS_F)
            prefix = pfx(prefix, of, comm_cf, hop)
            if hop <= B_HOPS:
                ob = origin_b(hop)
                accum(ob, comm_b, hop % SLOTS_B)
                prefix = pfx(prefix, ob, comm_cb, hop)

        my_prefix = jnp.sum(oh * prefix, axis=1, keepdims=True)
        accept = ((my_prefix + rank) < CAPACITY).astype(jnp.float32)
        out_ref[:, :] *= accept

        for pair in fwd + bwd:
            for r_ in pair:
                r_.wait_send()

        @functools.partial(pl.run_scoped,
                           second_barrier=pltpu.SemaphoreType.REGULAR)
        def _(second_barrier):
            for nbr in (left, right):
                pl.semaphore_signal(second_barrier, inc=1, device_id=(nbr,),
                                    device_id_type=pl.DeviceIdType.MESH)
            pl.semaphore_wait(second_barrier, 2)

    return pl.pallas_call(
        body,
        out_shape=jax.ShapeDtypeStruct((tokens, h), jnp.float32),
        in_specs=[pl.BlockSpec(memory_space=pltpu.VMEM)] * 3
        + [pl.BlockSpec(memory_space=pltpu.SMEM)],
        out_specs=pl.BlockSpec(memory_space=pltpu.VMEM),
        scratch_shapes=[
            pltpu.VMEM((SLOTS_F, e_loc, d, h), jnp.bfloat16),
            pltpu.VMEM((SLOTS_B, e_loc, d, h), jnp.bfloat16),
            pltpu.VMEM((F_HOPS + 1, 8, N_EXPERTS), jnp.int32),
            pltpu.VMEM((B_HOPS + 1, 8, N_EXPERTS), jnp.int32),
            pltpu.SemaphoreType.DMA((F_HOPS,)),
            pltpu.SemaphoreType.DMA((F_HOPS,)),
            pltpu.SemaphoreType.DMA((B_HOPS,)),
            pltpu.SemaphoreType.DMA((B_HOPS,)),
            pltpu.SemaphoreType.DMA((F_HOPS,)),
            pltpu.SemaphoreType.DMA((F_HOPS,)),
            pltpu.SemaphoreType.DMA((B_HOPS,)),
            pltpu.SemaphoreType.DMA((B_HOPS,)),
        ],
        compiler_params=pltpu.CompilerParams(collective_id=0),
    )(x, route_idx, expert_W, meta)


# device time: 201465 ns/iter; 2.2260x vs baseline; 1.1317x over previous
import functools

import jax
import jax.numpy as jnp
from jax import lax
from jax.experimental import pallas as pl
from jax.experimental.pallas import tpu as pltpu

N_DEV = 32
N_EXPERTS = 128
CAPACITY = 102.0
F_HOPS = 16
B_HOPS = 15
SLOTS_F = 9
SLOTS_B = 8
CHUNKS = 2

_POS = {}
_p = 0
for _z in range(4):
    for _y in range(4):
        for _x in ([0, 1] if _y % 2 == 0 else [1, 0]):
            _POS[(_x, _y, _z)] = _p
            _p += 1

_SNAKE_YZ = [(0, 0), (1, 0), (2, 0), (3, 0), (3, 1), (2, 1), (1, 1), (0, 1),
             (0, 2), (1, 2), (2, 2), (3, 2), (3, 3), (2, 3), (1, 3), (0, 3)]
_CYC = ([(0, y, z) for (y, z) in _SNAKE_YZ]
        + [(1, y, z) for (y, z) in reversed(_SNAKE_YZ)])
RING = [_POS[c] for c in _CYC]
SIGMA = [RING.index(m) for m in range(N_DEV)]


def kernel(x, router_W, route_idx, expert_W):
    del router_W
    tokens, d = x.shape
    e_loc, _, h = expert_W.shape

    my = lax.axis_index("i")
    ring = jnp.asarray(RING, jnp.int32)
    sigma = jnp.asarray(SIGMA, jnp.int32)
    ci = sigma[my]
    nxt = ring[(ci + 1) % N_DEV]
    prv = ring[(ci - 1) % N_DEV]
    ofs = ring[(ci - jnp.arange(1, F_HOPS + 1)) % N_DEV]
    obs = ring[(ci + jnp.arange(1, B_HOPS + 1)) % N_DEV]
    meta = jnp.concatenate(
        [jnp.stack([nxt, prv]), ofs, obs]).astype(jnp.int32).reshape(1, -1)

    def body(x_ref, ridx_ref, ew_ref, meta_ref, out_ref,
             comm_f, comm_b, comm_cf, comm_cb,
             sw_f, rw_f, sw_b, rw_b, sc_f, rc_f, sc_b, rc_b):
        my = lax.axis_index("i")
        right = meta_ref[0, 0]
        left = meta_ref[0, 1]

        def origin_f(hop):
            return meta_ref[0, 2 + (hop - 1)]

        def origin_b(hop):
            return meta_ref[0, 2 + F_HOPS + (hop - 1)]

        barrier = pltpu.get_barrier_semaphore()
        for nbr in (left, right):
            pl.semaphore_signal(barrier, inc=1, device_id=(nbr,),
                                device_id_type=pl.DeviceIdType.MESH)
        pl.semaphore_wait(barrier, 2)

        x_bf = x_ref[:, :].astype(jnp.bfloat16)
        route = ridx_ref[:, :]

        eids = lax.broadcasted_iota(jnp.int32, (tokens, N_EXPERTS), 1)
        oh = (route == eids).astype(jnp.float32)
        row = lax.broadcasted_iota(jnp.int32, (tokens, tokens), 0)
        col = lax.broadcasted_iota(jnp.int32, (tokens, tokens), 1)
        lower = (col < row).astype(jnp.float32)
        csum_excl = jnp.dot(lower, oh, preferred_element_type=jnp.float32)
        rank = jnp.sum(csum_excl * oh, axis=1, keepdims=True)
        counts = jnp.sum(oh, axis=0, keepdims=True)

        own_w = ew_ref[:, :, :].astype(jnp.bfloat16)
        comm_f[0, :, :, :] = own_w
        comm_b[0, :, :, :] = own_w
        cbcast = jnp.broadcast_to(counts.astype(jnp.int32), comm_cf.shape[1:])
        comm_cf[0, :, :] = cbcast
        comm_cb[0, :, :] = cbcast

        out_ref[:, :] = jnp.zeros((tokens, h), jnp.float32)

        def accum(origin, comm, slot):
            for j in range(e_loc):
                e = origin * e_loc + j
                m = (route == e).astype(jnp.bfloat16)
                out_ref[:, :] += jnp.dot(
                    x_bf * m, comm[slot, j, :, :],
                    preferred_element_type=jnp.float32)

        def pfx(prefix, origin, comm_c, slot):
            cnts = comm_c[slot, 0:1, :].astype(jnp.float32)
            return prefix + jnp.where(origin < my, cnts, 0.0)

        epc = e_loc // CHUNKS

        def mk(comm, comm_c, ssem, rsem, csem, crsem, dst, hop, n_slots):
            rws = [pltpu.make_async_remote_copy(
                src_ref=comm.at[(hop - 1) % n_slots, pl.ds(k * epc, epc)],
                dst_ref=comm.at[hop % n_slots, pl.ds(k * epc, epc)],
                send_sem=ssem.at[(hop - 1) * CHUNKS + k],
                recv_sem=rsem.at[(hop - 1) * CHUNKS + k],
                device_id=(dst,), device_id_type=pl.DeviceIdType.MESH)
                for k in range(CHUNKS)]
            rc = pltpu.make_async_remote_copy(
                src_ref=comm_c.at[hop - 1], dst_ref=comm_c.at[hop],
                send_sem=csem.at[hop - 1], recv_sem=crsem.at[hop - 1],
                device_id=(dst,), device_id_type=pl.DeviceIdType.MESH)
            return rws + [rc]

        fwd = [mk(comm_f, comm_cf, sw_f, rw_f, sc_f, rc_f, right, hp, SLOTS_F)
               for hp in range(1, F_HOPS + 1)]
        bwd = [mk(comm_b, comm_cb, sw_b, rw_b, sc_b, rc_b, left, hp, SLOTS_B)
               for hp in range(1, B_HOPS + 1)]

        for r_ in fwd[0] + bwd[0]:
            r_.start()

        accum(my, comm_f, 0)

        prefix = jnp.zeros((1, N_EXPERTS), jnp.float32)
        for hop in range(1, F_HOPS + 1):
            n_pieces = CHUNKS + 1
            for k in range(n_pieces):
                fwd[hop - 1][k].wait_recv()
                if hop < F_HOPS:
                    fwd[hop][k].start()
                if hop <= B_HOPS:
                    bwd[hop - 1][k].wait_recv()
                    if hop < B_HOPS:
                        bwd[hop][k].start()

            of = origin_f(hop)
            accum(of, comm_f, hop % SLOTS_F)
            prefix = pfx(prefix, of, comm_cf, hop)
            if hop <= B_HOPS:
                ob = origin_b(hop)
                accum(ob, comm_b, hop % SLOTS_B)
                prefix = pfx(prefix, ob, comm_cb, hop)

        my_prefix = jnp.sum(oh * prefix, axis=1, keepdims=True)
        accept = ((my_prefix + rank) < CAPACITY).astype(jnp.float32)
        out_ref[:, :] *= accept

        for group in fwd + bwd:
            for r_ in group:
                r_.wait_send()

        @functools.partial(pl.run_scoped,
                           second_barrier=pltpu.SemaphoreType.REGULAR)
        def _(second_barrier):
            for nbr in (left, right):
                pl.semaphore_signal(second_barrier, inc=1, device_id=(nbr,),
                                    device_id_type=pl.DeviceIdType.MESH)
            pl.semaphore_wait(second_barrier, 2)

    return pl.pallas_call(
        body,
        out_shape=jax.ShapeDtypeStruct((tokens, h), jnp.float32),
        in_specs=[pl.BlockSpec(memory_space=pltpu.VMEM)] * 3
        + [pl.BlockSpec(memory_space=pltpu.SMEM)],
        out_specs=pl.BlockSpec(memory_space=pltpu.VMEM),
        scratch_shapes=[
            pltpu.VMEM((SLOTS_F, e_loc, d, h), jnp.bfloat16),
            pltpu.VMEM((SLOTS_B, e_loc, d, h), jnp.bfloat16),
            pltpu.VMEM((F_HOPS + 1, 8, N_EXPERTS), jnp.int32),
            pltpu.VMEM((B_HOPS + 1, 8, N_EXPERTS), jnp.int32),
            pltpu.SemaphoreType.DMA((F_HOPS * CHUNKS,)),
            pltpu.SemaphoreType.DMA((F_HOPS * CHUNKS,)),
            pltpu.SemaphoreType.DMA((B_HOPS * CHUNKS,)),
            pltpu.SemaphoreType.DMA((B_HOPS * CHUNKS,)),
            pltpu.SemaphoreType.DMA((F_HOPS,)),
            pltpu.SemaphoreType.DMA((F_HOPS,)),
            pltpu.SemaphoreType.DMA((B_HOPS,)),
            pltpu.SemaphoreType.DMA((B_HOPS,)),
        ],
        compiler_params=pltpu.CompilerParams(collective_id=0),
    )(x, route_idx, expert_W, meta)
